# baseline (speedup 1.0000x reference)
import numpy as np
import jax
import jax.numpy as jnp
from jax import lax
from jax.experimental import pallas as pl
from jax.experimental.pallas import tpu as pltpu
from jax.experimental.pallas import tpu_sc as plsc

_VOCAB = 100000
_D = 512
_SEQ = 8192
_LANES = 16

_NC = 2
_NS = 16
_NW = _NC * _NS
_BPW = _SEQ // _NW
_C = 32
_NCHUNK = _BPW // _C
_M = _SEQ // _C


def _pe_factors():
    col = np.arange(_D, dtype=np.float64)
    w = 1.0 / np.power(10000.0, 2.0 * np.floor(col / 2.0) / _D)
    m = np.arange(_M, dtype=np.float64)[:, None]
    r = np.arange(_C, dtype=np.float64)[:, None]
    even = (np.arange(_D) % 2 == 0)
    a = np.where(even, np.sin(m * _C * w), np.cos(m * _C * w))
    b = np.where(even, np.cos(m * _C * w), -np.sin(m * _C * w))
    u = np.cos(r * w)
    v = np.sin(r * w)
    return (a.astype(np.float32), b.astype(np.float32),
            u.astype(np.float32), v.astype(np.float32))


_A, _B, _U, _V = _pe_factors()


def _body(x_hbm, table_hbm, a_hbm, b_hbm, u_hbm, v_hbm, out_hbm,
          idx_v, rows_v, a_v, b_v, u_v, v_v, sem_g, sem_w):
    cid = lax.axis_index("c")
    sid = lax.axis_index("s")
    wid = sid * _NC + cid
    base0 = wid * _BPW
    m0 = wid * _NCHUNK

    pltpu.sync_copy(x_hbm.at[pl.ds(base0, _BPW)], idx_v)
    pltpu.sync_copy(a_hbm.at[pl.ds(m0, _NCHUNK)], a_v)
    pltpu.sync_copy(b_hbm.at[pl.ds(m0, _NCHUNK)], b_v)
    pltpu.sync_copy(u_hbm, u_v)
    pltpu.sync_copy(v_hbm, v_v)

    def start_gather(j):
        b = j % 2
        return pltpu.async_copy(table_hbm.at[idx_v.at[pl.ds(j * _C, _C)]],
                                rows_v.at[b], sem_g.at[b])

    inflight = {0: start_gather(0)}
    wb = {}
    for j in range(_NCHUNK):
        b = j % 2
        if j - 1 >= 0:
            wb[j - 1].wait()
        if j + 1 < _NCHUNK:
            inflight[j + 1] = start_gather(j + 1)
        inflight.pop(j).wait()

        def add_cols(k):
            sl = pl.ds(k * _LANES, _LANES)
            av = a_v[j, sl]
            bv = b_v[j, sl]
            for r in range(_C):
                rows_v[b, r, sl] = (rows_v[b, r, sl]
                                    + av * u_v[r, sl] + bv * v_v[r, sl])

        lax.fori_loop(0, _D // _LANES, lambda k, _: (add_cols(k), 0)[1], 0)

        wb[j] = pltpu.async_copy(rows_v.at[b],
                                 out_hbm.at[pl.ds(base0 + j * _C, _C)],
                                 sem_w.at[b])
    wb[_NCHUNK - 1].wait()


_sc_call = pl.kernel(
    _body,
    out_type=jax.ShapeDtypeStruct((_SEQ, _D), jnp.float32),
    mesh=plsc.VectorSubcoreMesh(core_axis_name="c", subcore_axis_name="s"),
    scratch_types=[
        pltpu.VMEM((_BPW,), jnp.int32),
        pltpu.VMEM((2, _C, _D), jnp.float32),
        pltpu.VMEM((_NCHUNK, _D), jnp.float32),
        pltpu.VMEM((_NCHUNK, _D), jnp.float32),
        pltpu.VMEM((_C, _D), jnp.float32),
        pltpu.VMEM((_C, _D), jnp.float32),
        pltpu.SemaphoreType.DMA((2,)),
        pltpu.SemaphoreType.DMA((2,)),
    ],
)


def kernel(x, table):
    xi = x.astype(jnp.int32)
    return _sc_call(xi, table, jnp.asarray(_A), jnp.asarray(_B),
                    jnp.asarray(_U), jnp.asarray(_V))


# speedup vs baseline: 1.0396x; 1.0396x over previous
import numpy as np
import jax
import jax.numpy as jnp
from jax import lax
from jax.experimental import pallas as pl
from jax.experimental.pallas import tpu as pltpu
from jax.experimental.pallas import tpu_sc as plsc

_VOCAB = 100000
_D = 512
_SEQ = 8192
_LANES = 16

_NC = 2
_NS = 16
_NW = _NC * _NS
_BPW = _SEQ // _NW
_C = 32
_NCHUNK = _BPW // _C
_M = _SEQ // _C


def _pe_factors():
    col = np.arange(_D, dtype=np.float64)
    w = 1.0 / np.power(10000.0, 2.0 * np.floor(col / 2.0) / _D)
    m = np.arange(_M, dtype=np.float64)[:, None]
    r = np.arange(_C, dtype=np.float64)[:, None]
    even = (np.arange(_D) % 2 == 0)
    a = np.where(even, np.sin(m * _C * w), np.cos(m * _C * w))
    b = np.where(even, np.cos(m * _C * w), -np.sin(m * _C * w))
    u = np.cos(r * w)
    v = np.sin(r * w)
    return (a.astype(np.float32), b.astype(np.float32),
            u.astype(np.float32), v.astype(np.float32))


_A, _B, _U, _V = _pe_factors()


def _body(x_hbm, table_hbm, a_hbm, b_hbm, u_hbm, v_hbm, out_hbm,
          idx_v, rows_v, a_v, b_v, u_v, v_v, sem_g, sem_w):
    cid = lax.axis_index("c")
    sid = lax.axis_index("s")
    wid = sid * _NC + cid
    base0 = wid * _BPW
    m0 = wid * _NCHUNK

    pltpu.sync_copy(x_hbm.at[pl.ds(base0, _BPW)], idx_v)
    pltpu.sync_copy(a_hbm.at[pl.ds(m0, _NCHUNK)], a_v)
    pltpu.sync_copy(b_hbm.at[pl.ds(m0, _NCHUNK)], b_v)
    pltpu.sync_copy(u_hbm, u_v)
    pltpu.sync_copy(v_hbm, v_v)

    def start_gather(j):
        b = j % 2
        return pltpu.async_copy(table_hbm.at[idx_v.at[pl.ds(j * _C, _C)]],
                                rows_v.at[b], sem_g.at[b])

    inflight = {0: start_gather(0)}
    wb = {}
    for j in range(_NCHUNK):
        b = j % 2
        if j - 1 >= 0:
            wb[j - 1].wait()
        if j + 1 < _NCHUNK:
            inflight[j + 1] = start_gather(j + 1)
        inflight.pop(j).wait()

        @plsc.parallel_loop(0, _D // _LANES, unroll=2)
        def _add_cols(k):
            sl = pl.ds(k * _LANES, _LANES)
            av = a_v[j, sl]
            bv = b_v[j, sl]
            for r in range(_C):
                rows_v[b, r, sl] = (rows_v[b, r, sl]
                                    + av * u_v[r, sl] + bv * v_v[r, sl])

        wb[j] = pltpu.async_copy(rows_v.at[b],
                                 out_hbm.at[pl.ds(base0 + j * _C, _C)],
                                 sem_w.at[b])
    wb[_NCHUNK - 1].wait()


_sc_call = pl.kernel(
    _body,
    out_type=jax.ShapeDtypeStruct((_SEQ, _D), jnp.float32),
    mesh=plsc.VectorSubcoreMesh(core_axis_name="c", subcore_axis_name="s"),
    scratch_types=[
        pltpu.VMEM((_BPW,), jnp.int32),
        pltpu.VMEM((2, _C, _D), jnp.float32),
        pltpu.VMEM((_NCHUNK, _D), jnp.float32),
        pltpu.VMEM((_NCHUNK, _D), jnp.float32),
        pltpu.VMEM((_C, _D), jnp.float32),
        pltpu.VMEM((_C, _D), jnp.float32),
        pltpu.SemaphoreType.DMA((2,)),
        pltpu.SemaphoreType.DMA((2,)),
    ],
)


def kernel(x, table):
    xi = x.astype(jnp.int32)
    return _sc_call(xi, table, jnp.asarray(_A), jnp.asarray(_B),
                    jnp.asarray(_U), jnp.asarray(_V))


# speedup vs baseline: 1.1332x; 1.0900x over previous
import ml_dtypes
import numpy as np
import jax
import jax.numpy as jnp
from jax import lax
from jax.experimental import pallas as pl
from jax.experimental.pallas import tpu as pltpu
from jax.experimental.pallas import tpu_sc as plsc

_VOCAB = 100000
_D = 512
_SEQ = 8192
_LANES = 16

_NC = 2
_NS = 16
_NW = _NC * _NS
_BPW = _SEQ // _NW
_C = 32
_NCHUNK = _BPW // _C
_M = _SEQ // _C


def _pe_factors():
    col = np.arange(_D, dtype=np.float64)
    w = 1.0 / np.power(10000.0, 2.0 * np.floor(col / 2.0) / _D)
    m = np.arange(_M, dtype=np.float64)[:, None]
    r = np.arange(_C, dtype=np.float64)[:, None]
    even = (np.arange(_D) % 2 == 0)
    ab = np.stack([np.where(even, np.sin(m * _C * w), np.cos(m * _C * w)),
                   np.where(even, np.cos(m * _C * w), -np.sin(m * _C * w))])
    uv = np.stack([np.cos(r * w), np.sin(r * w)])
    return ab.astype(np.float32), uv.astype(np.float32)


_AB, _UV = _pe_factors()


def _body(x_hbm, table_hbm, ab_hbm, uv_hbm, out_hbm,
          idx_v, rows_v, out_v, ab_v, uv_v, sem_g, sem_w):
    cid = lax.axis_index("c")
    sid = lax.axis_index("s")
    wid = sid * _NC + cid
    base0 = wid * _BPW
    m0 = wid * _NCHUNK

    pltpu.sync_copy(x_hbm.at[pl.ds(base0, _BPW)], idx_v)
    pltpu.sync_copy(ab_hbm.at[0, pl.ds(m0, _NCHUNK)], ab_v.at[0])
    pltpu.sync_copy(ab_hbm.at[1, pl.ds(m0, _NCHUNK)], ab_v.at[1])
    pltpu.sync_copy(uv_hbm, uv_v)

    def start_gather(j):
        b = j % 2
        return pltpu.async_copy(table_hbm.at[idx_v.at[pl.ds(j * _C, _C)]],
                                rows_v.at[b], sem_g.at[b])

    inflight = {0: start_gather(0)}
    wb = {}
    for j in range(_NCHUNK):
        b = j % 2
        if j - 1 >= 0:
            wb[j - 1].wait()
        if j + 1 < _NCHUNK:
            inflight[j + 1] = start_gather(j + 1)
        inflight.pop(j).wait()

        @plsc.parallel_loop(0, _D // _LANES)
        def _add_cols(k):
            sl = pl.ds(k * _LANES, _LANES)
            av = ab_v[0, j, sl]
            bv = ab_v[1, j, sl]
            for r in range(_C):
                out_v[b, r, sl] = (rows_v[b, r, sl]
                                   + av * uv_v[0, r, sl]
                                   + bv * uv_v[1, r, sl])

        wb[j] = pltpu.async_copy(out_v.at[b],
                                 out_hbm.at[pl.ds(base0 + j * _C, _C)],
                                 sem_w.at[b])
    wb[_NCHUNK - 1].wait()


_sc_call = pl.kernel(
    _body,
    out_type=jax.ShapeDtypeStruct((_SEQ, _D), jnp.float32),
    mesh=plsc.VectorSubcoreMesh(core_axis_name="c", subcore_axis_name="s"),
    scratch_types=[
        pltpu.VMEM((_BPW,), jnp.int32),
        pltpu.VMEM((2, _C, _D), jnp.float32),
        pltpu.VMEM((2, _C, _D), jnp.float32),
        pltpu.VMEM((2, _NCHUNK, _D), jnp.float32),
        pltpu.VMEM((2, _C, _D), jnp.float32),
        pltpu.SemaphoreType.DMA((2,)),
        pltpu.SemaphoreType.DMA((2,)),
    ],
)


def kernel(x, table):
    xi = x.astype(jnp.int32)
    return _sc_call(xi, table, jnp.asarray(_AB), jnp.asarray(_UV))


# speedup vs baseline: 1.2272x; 1.0830x over previous
import ml_dtypes
import numpy as np
import jax
import jax.numpy as jnp
from jax import lax
from jax.experimental import pallas as pl
from jax.experimental.pallas import tpu as pltpu
from jax.experimental.pallas import tpu_sc as plsc

_VOCAB = 100000
_D = 512
_SEQ = 8192
_LANES = 16

_NC = 2
_NS = 16
_NW = _NC * _NS
_BPW = _SEQ // _NW
_C = 32
_NCHUNK = _BPW // _C
_H = 16
_MH = _SEQ // _H


def _pe_factors():
    col = np.arange(_D, dtype=np.float64)
    w = 1.0 / np.power(10000.0, 2.0 * np.floor(col / 2.0) / _D)
    m = np.arange(_MH, dtype=np.float64)[:, None]
    r = np.arange(_H, dtype=np.float64)[:, None]
    even = (np.arange(_D) % 2 == 0)
    ab = np.stack([np.where(even, np.sin(m * _H * w), np.cos(m * _H * w)),
                   np.where(even, np.cos(m * _H * w), -np.sin(m * _H * w))])
    uv = np.stack([np.cos(r * w), np.sin(r * w)])
    return ab.astype(np.float32), uv.astype(np.float32)


_AB, _UV = _pe_factors()


def _body(x_hbm, table_hbm, ab_hbm, uv_hbm, out_hbm,
          idx0, idx1, idx2, idx3, idx4, idx5, idx6, idx7,
          rows_v, out_v, ab_v, uv_v, sem_g, sem_w):
    cid = lax.axis_index("c")
    sid = lax.axis_index("s")
    wid = sid * _NC + cid
    base0 = wid * _BPW
    m0 = wid * (_BPW // _H)

    idx_bufs = [idx0, idx1, idx2, idx3, idx4, idx5, idx6, idx7]

    for j in range(_NCHUNK):
        pltpu.sync_copy(x_hbm.at[pl.ds(base0 + j * _C, _C)], idx_bufs[j])
    pltpu.sync_copy(ab_hbm.at[0, pl.ds(m0, 2 * _NCHUNK)], ab_v.at[0])
    pltpu.sync_copy(ab_hbm.at[1, pl.ds(m0, 2 * _NCHUNK)], ab_v.at[1])
    pltpu.sync_copy(uv_hbm, uv_v)

    def start_gather(j):
        b = j % 2
        return pltpu.async_copy(table_hbm.at[idx_bufs[j]], rows_v.at[b],
                                sem_g.at[b])

    inflight = {0: start_gather(0)}
    wb = {}
    for j in range(_NCHUNK):
        b = j % 2
        if j - 1 >= 0:
            wb[j - 1].wait()
        if j + 1 < _NCHUNK:
            inflight[j + 1] = start_gather(j + 1)
        inflight.pop(j).wait()

        @plsc.parallel_loop(0, _D // _LANES)
        def _add_cols(k):
            sl = pl.ds(k * _LANES, _LANES)
            av0 = ab_v[0, 2 * j, sl]
            bv0 = ab_v[1, 2 * j, sl]
            av1 = ab_v[0, 2 * j + 1, sl]
            bv1 = ab_v[1, 2 * j + 1, sl]
            for r in range(_H):
                u = uv_v[0, r, sl]
                v = uv_v[1, r, sl]
                out_v[b, r, sl] = (rows_v[b, r, sl] + av0 * u + bv0 * v)
                out_v[b, _H + r, sl] = (rows_v[b, _H + r, sl]
                                        + av1 * u + bv1 * v)

        wb[j] = pltpu.async_copy(out_v.at[b],
                                 out_hbm.at[pl.ds(base0 + j * _C, _C)],
                                 sem_w.at[b])
    wb[_NCHUNK - 1].wait()


_sc_call = pl.kernel(
    _body,
    out_type=jax.ShapeDtypeStruct((_SEQ, _D), jnp.float32),
    mesh=plsc.VectorSubcoreMesh(core_axis_name="c", subcore_axis_name="s"),
    scratch_types=(
        [pltpu.VMEM((_C,), jnp.int32) for _ in range(_NCHUNK)] + [
            pltpu.VMEM((2, _C, _D), jnp.float32),
            pltpu.VMEM((2, _C, _D), jnp.float32),
            pltpu.VMEM((2, 2 * _NCHUNK, _D), jnp.float32),
            pltpu.VMEM((2, _H, _D), jnp.float32),
            pltpu.SemaphoreType.DMA((2,)),
            pltpu.SemaphoreType.DMA((2,)),
        ]),
)


def kernel(x, table):
    xi = x.astype(jnp.int32)
    return _sc_call(xi, table, jnp.asarray(_AB), jnp.asarray(_UV))


# speedup vs baseline: 1.3493x; 1.0995x over previous
import ml_dtypes
import numpy as np
import jax
import jax.numpy as jnp
from jax import lax
from jax.experimental import pallas as pl
from jax.experimental.pallas import tpu as pltpu
from jax.experimental.pallas import tpu_sc as plsc

_VOCAB = 100000
_D = 512
_SEQ = 8192
_LANES = 16

_NC = 2
_NS = 16
_NW = _NC * _NS
_BPW = _SEQ // _NW
_C = 32
_NCHUNK = _BPW // _C
_H = 16
_MH = _SEQ // _H


def _pe_factors():
    col = np.arange(_D, dtype=np.float64)
    w = 1.0 / np.power(10000.0, 2.0 * np.floor(col / 2.0) / _D)
    m = np.arange(_MH, dtype=np.float64)[:, None]
    r = np.arange(_H, dtype=np.float64)[:, None]
    even = (np.arange(_D) % 2 == 0)
    ab = np.stack([np.where(even, np.sin(m * _H * w), np.cos(m * _H * w)),
                   np.where(even, np.cos(m * _H * w), -np.sin(m * _H * w))])
    uv = np.stack([np.cos(r * w), np.sin(r * w)])
    return np.concatenate([ab, uv], axis=1).astype(np.float32)


_ABUV = _pe_factors()


def _body(x_hbm, table_hbm, abuv_hbm, out_hbm,
          idx_v, rows_v, out_v, ab_v, uv_v, sem_g, sem_w, sem_f):
    cid = lax.axis_index("c")
    sid = lax.axis_index("s")
    wid = sid * _NC + cid
    base0 = wid * _BPW
    m0 = wid * (_BPW // _H)

    pltpu.sync_copy(x_hbm.at[pl.ds(base0, _BPW)], idx_v)
    f0 = pltpu.async_copy(abuv_hbm.at[0, pl.ds(m0, 2 * _NCHUNK)],
                          ab_v.at[0], sem_f.at[0])
    f1 = pltpu.async_copy(abuv_hbm.at[1, pl.ds(m0, 2 * _NCHUNK)],
                          ab_v.at[1], sem_f.at[1])
    f2 = pltpu.async_copy(abuv_hbm.at[:, pl.ds(_MH, _H)], uv_v,
                          sem_f.at[2])

    def start_gather(j):
        b = j % 2
        return pltpu.async_copy(table_hbm.at[idx_v.at[pl.ds(j * _C, _C)]],
                                rows_v.at[b], sem_g.at[b])

    inflight = {0: start_gather(0)}
    f0.wait()
    f1.wait()
    f2.wait()
    wb = {}
    for j in range(_NCHUNK):
        b = j % 2
        if j - 1 >= 0:
            wb[j - 1].wait()
        if j + 1 < _NCHUNK:
            inflight[j + 1] = start_gather(j + 1)
        inflight.pop(j).wait()

        @plsc.parallel_loop(0, _D // _LANES)
        def _add_cols(k):
            sl = pl.ds(k * _LANES, _LANES)
            av0 = ab_v[0, 2 * j, sl]
            bv0 = ab_v[1, 2 * j, sl]
            av1 = ab_v[0, 2 * j + 1, sl]
            bv1 = ab_v[1, 2 * j + 1, sl]
            for r in range(_H):
                u = uv_v[0, r, sl]
                v = uv_v[1, r, sl]
                out_v[b, r, sl] = (rows_v[b, r, sl] + av0 * u + bv0 * v)
                out_v[b, _H + r, sl] = (rows_v[b, _H + r, sl]
                                        + av1 * u + bv1 * v)

        wb[j] = pltpu.async_copy(out_v.at[b],
                                 out_hbm.at[pl.ds(base0 + j * _C, _C)],
                                 sem_w.at[b])
    wb[_NCHUNK - 1].wait()


_sc_call = pl.kernel(
    _body,
    out_type=jax.ShapeDtypeStruct((_SEQ, _D), jnp.float32),
    mesh=plsc.VectorSubcoreMesh(core_axis_name="c", subcore_axis_name="s"),
    scratch_types=[
        pltpu.VMEM((_BPW,), jnp.int32),
        pltpu.VMEM((2, _C, _D), jnp.float32),
        pltpu.VMEM((2, _C, _D), jnp.float32),
        pltpu.VMEM((2, 2 * _NCHUNK, _D), jnp.float32),
        pltpu.VMEM((2, _H, _D), jnp.float32),
        pltpu.SemaphoreType.DMA((2,)),
        pltpu.SemaphoreType.DMA((2,)),
        pltpu.SemaphoreType.DMA((3,)),
    ],
)


def kernel(x, table):
    xi = x.astype(jnp.int32)
    return _sc_call(xi, table, jnp.asarray(_ABUV))
